# baseline (speedup 1.0000x reference)
import functools

import jax
import jax.numpy as jnp
from jax import lax
from jax.experimental import pallas as pl
from jax.experimental.pallas import tpu as pltpu
from jax.experimental.pallas import tpu_sc as plsc

B, S, D = 16, 2048, 1024
HALF = S // 2
CHUNK = 32
NDK = D // 16
NCH = HALF // CHUNK


def _allsum(v):
    for k in (8, 4, 2, 1):
        idx = (lax.iota(jnp.int32, 16) + k) & 15
        v = v + v.at[idx].get(mode="promise_in_bounds")
    return v


def _sc_partials(x2, lens, w1d):
    mesh = plsc.VectorSubcoreMesh(core_axis_name="c", subcore_axis_name="s")

    @functools.partial(
        pl.kernel, mesh=mesh,
        out_type=(jax.ShapeDtypeStruct((2, B, D), jnp.float32),
                  jax.ShapeDtypeStruct((2, B, 16), jnp.float32)),
        scratch_types=[
            pltpu.VMEM((CHUNK, D), jnp.float32),
            pltpu.VMEM((D,), jnp.float32),
            pltpu.VMEM((D,), jnp.float32),
            pltpu.VMEM((16,), jnp.int32),
            pltpu.VMEM((16,), jnp.float32),
        ],
    )
    def k(x_hbm, lens_hbm, w_hbm, acc_out, l_out, xbuf, wbuf, accbuf,
          lensbuf, lbuf):
        c = lax.axis_index("c")
        s = lax.axis_index("s")
        pltpu.sync_copy(lens_hbm, lensbuf)
        pltpu.sync_copy(w_hbm, wbuf)
        sel = lax.iota(jnp.int32, 16) == jnp.full((16,), s, jnp.int32)
        len_vec = _allsum(jnp.where(sel, lensbuf[...].astype(jnp.float32),
                                    0.0))
        tcnt_vec = jnp.clip(len_vec - jnp.full((16,), (c * HALF) * 1.0),
                            0.0, float(HALF))
        base = s * S + c * HALF
        for dk in range(NDK):
            accbuf[pl.ds(dk * 16, 16)] = jnp.zeros((16,), jnp.float32)

        def chunk_body(kk, lsum):
            pltpu.sync_copy(x_hbm.at[pl.ds(base + kk * CHUNK, CHUNK)], xbuf)

            def token_body(t, lsum_t):
                lacc = jnp.zeros((16,), jnp.float32)
                for dk in range(NDK):
                    lacc = lacc + (xbuf[t, pl.ds(dk * 16, 16)]
                                   * wbuf[pl.ds(dk * 16, 16)])
                logit_vec = _allsum(lacc)
                tok = (kk * CHUNK + t).astype(jnp.float32)
                valid = jnp.full((16,), tok) < tcnt_vec
                wv = jnp.exp(jnp.where(valid, logit_vec,
                                       jnp.full((16,), -jnp.inf)))
                for dk in range(NDK):
                    sl = pl.ds(dk * 16, 16)
                    accbuf[sl] = accbuf[sl] + wv * xbuf[t, sl]
                return lsum_t + wv

            return lax.fori_loop(0, CHUNK, token_body, lsum)

        lsum = lax.fori_loop(0, NCH, chunk_body,
                             jnp.zeros((16,), jnp.float32))
        lbuf[...] = lsum
        pltpu.sync_copy(accbuf, acc_out.at[c, s])
        pltpu.sync_copy(lbuf, l_out.at[c, s])

    return k(x2, lens, w1d)


def _merge_body(acc_ref, l_ref, o_ref):
    l_tot = l_ref[0, :, 0:1] + l_ref[1, :, 0:1]
    o_ref[...] = (acc_ref[0] + acc_ref[1]) / l_tot


def kernel(x, x_lens, W, b):
    lens = x_lens.astype(jnp.int32)
    x2 = x.reshape(B * S, D)
    w1d = W.reshape(D)
    acc_p, l_p = _sc_partials(x2, lens, w1d)
    return pl.pallas_call(
        _merge_body,
        out_shape=jax.ShapeDtypeStruct((B, D), jnp.float32),
    )(acc_p, l_p)


# speedup vs baseline: 9.1999x; 9.1999x over previous
import jax
import jax.numpy as jnp
from jax.experimental import pallas as pl
from jax.experimental.pallas import tpu as pltpu

S_BLK = 128


def _body(x_ref, lens_ref, w_ref, o_ref, m_ref, l_ref, acc_ref):
    j = pl.program_id(0)
    nsteps = pl.num_programs(0)
    B, _, D = x_ref.shape

    @pl.when(j == 0)
    def _init():
        m_ref[...] = jnp.full_like(m_ref, -jnp.inf)
        l_ref[...] = jnp.zeros_like(l_ref)
        acc_ref[...] = jnp.zeros_like(acc_ref)

    xb = x_ref[...]
    xflat = xb.reshape(B * S_BLK, D)
    logits_flat = jax.lax.dot_general(
        xflat, w_ref[...], (((1,), (0,)), ((), ())),
        preferred_element_type=jnp.float32)
    logits = logits_flat.reshape(B, S_BLK, 1)
    pos = j * S_BLK + jax.lax.broadcasted_iota(jnp.int32, (B, S_BLK, 1), 1)
    mask = pos < lens_ref[...][:, :, None]
    logits = jnp.where(mask, logits, -jnp.inf)
    m_prev = m_ref[...]
    m_new = jnp.maximum(m_prev, jnp.max(logits, axis=1))
    alpha = jnp.exp(m_prev - m_new)
    alpha = jnp.where(m_new == -jnp.inf, 0.0, alpha)
    p = jnp.where(mask, jnp.exp(logits - m_new[:, :, None]), 0.0)
    l_ref[...] = l_ref[...] * alpha + jnp.sum(p, axis=1)
    m_ref[...] = m_new
    p2 = p.reshape(B, S_BLK)
    px = jax.lax.dot_general(
        p2, xb, (((1,), (1,)), ((0,), (0,))),
        preferred_element_type=jnp.float32)
    acc_ref[...] = acc_ref[...] * alpha + px

    @pl.when(j == nsteps - 1)
    def _fin():
        o_ref[...] = acc_ref[...] / l_ref[...]


def kernel(x, x_lens, W, b):
    B, S, D = x.shape
    lens2 = x_lens.astype(jnp.int32).reshape(B, 1)
    grid = (S // S_BLK,)
    return pl.pallas_call(
        _body,
        grid=grid,
        in_specs=[
            pl.BlockSpec((B, S_BLK, D), lambda j: (0, j, 0)),
            pl.BlockSpec((B, 1), lambda j: (0, 0)),
            pl.BlockSpec((D, 1), lambda j: (0, 0)),
        ],
        out_specs=pl.BlockSpec((B, D), lambda j: (0, 0)),
        scratch_shapes=[
            pltpu.VMEM((B, 1), jnp.float32),
            pltpu.VMEM((B, 1), jnp.float32),
            pltpu.VMEM((B, D), jnp.float32),
        ],
        out_shape=jax.ShapeDtypeStruct((B, D), jnp.float32),
        compiler_params=pltpu.CompilerParams(
            dimension_semantics=("arbitrary",)),
    )(x, lens2, W)


# speedup vs baseline: 9.6460x; 1.0485x over previous
import jax
import jax.numpy as jnp
from jax.experimental import pallas as pl
from jax.experimental.pallas import tpu as pltpu

S_BLK = 256


def _body(x_ref, lens_ref, w_ref, o_ref, m_ref, l_ref, acc_ref):
    j = pl.program_id(0)
    nsteps = pl.num_programs(0)
    B, _, D = x_ref.shape

    @pl.when(j == 0)
    def _init():
        m_ref[...] = jnp.full_like(m_ref, -jnp.inf)
        l_ref[...] = jnp.zeros_like(l_ref)
        acc_ref[...] = jnp.zeros_like(acc_ref)

    xb = x_ref[...]
    xflat = xb.reshape(B * S_BLK, D)
    logits_flat = jax.lax.dot_general(
        xflat, w_ref[...], (((1,), (0,)), ((), ())),
        preferred_element_type=jnp.float32)
    logits = logits_flat.reshape(B, S_BLK, 1)
    pos = j * S_BLK + jax.lax.broadcasted_iota(jnp.int32, (B, S_BLK, 1), 1)
    mask = pos < lens_ref[...][:, :, None]
    logits = jnp.where(mask, logits, -jnp.inf)
    m_prev = m_ref[...]
    m_new = jnp.maximum(m_prev, jnp.max(logits, axis=1))
    alpha = jnp.exp(m_prev - m_new)
    alpha = jnp.where(m_new == -jnp.inf, 0.0, alpha)
    p = jnp.where(mask, jnp.exp(logits - m_new[:, :, None]), 0.0)
    l_ref[...] = l_ref[...] * alpha + jnp.sum(p, axis=1)
    m_ref[...] = m_new
    p2 = p.reshape(B, S_BLK)
    px = jax.lax.dot_general(
        p2, xb, (((1,), (1,)), ((0,), (0,))),
        preferred_element_type=jnp.float32)
    acc_ref[...] = acc_ref[...] * alpha + px

    @pl.when(j == nsteps - 1)
    def _fin():
        o_ref[...] = acc_ref[...] / l_ref[...]


def kernel(x, x_lens, W, b):
    B, S, D = x.shape
    lens2 = x_lens.astype(jnp.int32).reshape(B, 1)
    grid = (S // S_BLK,)
    return pl.pallas_call(
        _body,
        grid=grid,
        in_specs=[
            pl.BlockSpec((B, S_BLK, D), lambda j: (0, j, 0)),
            pl.BlockSpec((B, 1), lambda j: (0, 0)),
            pl.BlockSpec((D, 1), lambda j: (0, 0)),
        ],
        out_specs=pl.BlockSpec((B, D), lambda j: (0, 0)),
        scratch_shapes=[
            pltpu.VMEM((B, 1), jnp.float32),
            pltpu.VMEM((B, 1), jnp.float32),
            pltpu.VMEM((B, D), jnp.float32),
        ],
        out_shape=jax.ShapeDtypeStruct((B, D), jnp.float32),
        compiler_params=pltpu.CompilerParams(
            dimension_semantics=("arbitrary",)),
    )(x, lens2, W)
